# baseline (speedup 1.0000x reference)
import functools

import jax
import jax.numpy as jnp
from jax import lax
from jax.experimental import pallas as pl
from jax.experimental.pallas import tpu as pltpu
from jax.experimental.pallas import tpu_sc as plsc

H = 128
BE = 1600
BN = 1000
CH = 128
CHG = 80
RB = 2000
F32 = jnp.float32


def _mm(a, b):
    return lax.dot_general(a.astype(jnp.bfloat16), b.astype(jnp.bfloat16),
                           (((1,), (0,)), ((), ())),
                           preferred_element_type=F32,
                           precision=lax.Precision.DEFAULT)


def _silu(x):
    return 0.5 * x * (1.0 + jnp.tanh(0.5 * x))


def _full(shape):
    return pl.BlockSpec(shape, lambda i: (0,) * len(shape))


def _row2(p):
    return p.reshape(1, -1)


def _make_sc_gather(E, N):
    info = plsc.get_sparse_core_info()
    NC, NS = info.num_cores, info.num_subcores
    NW = NC * NS
    ept = E // NW
    n_chunks = ept // CHG
    mesh = plsc.VectorSubcoreMesh(core_axis_name="c", subcore_axis_name="s")

    @functools.partial(
        pl.kernel,
        out_type=jax.ShapeDtypeStruct((E, H), F32),
        mesh=mesh,
        scratch_types=[
            pltpu.VMEM((ept,), jnp.int32), pltpu.VMEM((ept,), jnp.int32),
            pltpu.VMEM((CHG, H), F32), pltpu.VMEM((CHG, H), F32),
            pltpu.VMEM((CHG, H), F32), pltpu.VMEM((CHG, H), F32),
            pltpu.SemaphoreType.DMA, pltpu.SemaphoreType.DMA,
            pltpu.SemaphoreType.DMA, pltpu.SemaphoreType.DMA,
            pltpu.SemaphoreType.DMA, pltpu.SemaphoreType.DMA,
        ],
    )
    def gather_k(tab_r, tab_c, row_h, col_h, t_o,
                 idxr, idxc, br0, bc0, br1, bc1,
                 sgr0, sgc0, sgr1, sgc1, so0, so1):
        wid = lax.axis_index("s") * NC + lax.axis_index("c")
        e0 = wid * ept
        BRb, BCb = (br0, br1), (bc0, bc1)
        SGR, SGC, SO = (sgr0, sgr1), (sgc0, sgc1), (so0, so1)
        pltpu.sync_copy(row_h.at[pl.ds(e0, ept)], idxr)
        pltpu.sync_copy(col_h.at[pl.ds(e0, ept)], idxc)

        def issue(k, b):
            @pl.when((k >= 2) & (k - 2 < n_chunks))
            def _():
                pltpu.make_async_copy(
                    BRb[b], t_o.at[pl.ds(0, CHG)], SO[b]).wait()

            @pl.when(k < n_chunks)
            def _():
                off = k * CHG
                pltpu.async_copy(
                    tab_r.at[idxr.at[pl.ds(off, CHG)]], BRb[b], SGR[b])
                pltpu.async_copy(
                    tab_c.at[idxc.at[pl.ds(off, CHG)]], BCb[b], SGC[b])

        def drain(k, b):
            @pl.when((k >= 0) & (k < n_chunks))
            def _():
                off = k * CHG
                pltpu.make_async_copy(
                    tab_r.at[idxr.at[pl.ds(off, CHG)]], BRb[b], SGR[b]).wait()
                pltpu.make_async_copy(
                    tab_c.at[idxc.at[pl.ds(off, CHG)]], BCb[b], SGC[b]).wait()

                def rbody(r, carry):
                    for cix in range(H // 16):
                        sl = pl.ds(cix * 16, 16)
                        BRb[b][r, sl] = BRb[b][r, sl] + BCb[b][r, sl]
                    return carry

                lax.fori_loop(0, CHG, rbody, 0)
                pltpu.async_copy(BRb[b], t_o.at[pl.ds(e0 + off, CHG)], SO[b])

        def pair(k2, carry):
            for b2 in range(2):
                k = 2 * k2 + b2
                issue(k, b2)
                drain(k - 1, 1 - b2)
            return carry

        lax.fori_loop(0, (n_chunks + 4) // 2, pair, 0)

    return gather_k


def _make_sc_rel(E, N):
    info = plsc.get_sparse_core_info()
    NC, NS = info.num_cores, info.num_subcores
    NW = NC * NS
    ept = E // NW
    n_sup = ept // RB
    nv = RB // 16
    mesh = plsc.VectorSubcoreMesh(core_axis_name="c", subcore_axis_name="s")

    @functools.partial(
        pl.kernel,
        out_type=jax.ShapeDtypeStruct((E, 4), F32),
        mesh=mesh,
        compiler_params=pltpu.CompilerParams(use_tc_tiling_on_sc=False,
                                             needs_layout_passes=False),
        scratch_types=[
            pltpu.VMEM((N, 4), F32),
            pltpu.VMEM((RB,), jnp.int32), pltpu.VMEM((RB,), jnp.int32),
            pltpu.VMEM((RB, 4), F32), pltpu.VMEM((RB, 4), F32),
            pltpu.SemaphoreType.DMA, pltpu.SemaphoreType.DMA,
        ],
    )
    def rel_k(x4_h, row_h, col_h, rel_o, xtab, idxr, idxc, rb0, rb1, so0, so1):
        wid = lax.axis_index("s") * NC + lax.axis_index("c")
        e0 = wid * ept
        RBb, SO = (rb0, rb1), (so0, so1)
        pltpu.sync_copy(x4_h, xtab)
        lane = lax.iota(jnp.int32, 16)

        for s in range(n_sup):
            b = s % 2
            if s >= 2:
                pltpu.make_async_copy(
                    RBb[b], rel_o.at[pl.ds(0, RB)], SO[b]).wait()
            pltpu.sync_copy(row_h.at[pl.ds(e0 + s * RB, RB)], idxr)
            pltpu.sync_copy(col_h.at[pl.ds(e0 + s * RB, RB)], idxc)

            def vbody(v, carry):
                off = v * 16
                ir = idxr[pl.ds(off, 16)]
                ic = idxc[pl.ds(off, 16)]
                lrow = v * 16 + lane
                for j in range(3):
                    js = jnp.full((16,), j, jnp.int32)
                    xr = plsc.load_gather(xtab, [ir, js])
                    xc = plsc.load_gather(xtab, [ic, js])
                    plsc.store_scatter(RBb[b], [lrow, js], xr - xc)
                plsc.store_scatter(RBb[b], [lrow, jnp.full((16,), 3, jnp.int32)],
                                   jnp.zeros((16,), F32))
                return carry

            lax.fori_loop(0, nv, vbody, 0)
            pltpu.async_copy(RBb[b], rel_o.at[pl.ds(e0 + s * RB, RB)], SO[b])

        for s in (n_sup - 2, n_sup - 1):
            pltpu.make_async_copy(
                RBb[s % 2], rel_o.at[pl.ds(0, RB)], SO[s % 2]).wait()

    return rel_k


def _make_sc_scatter_m(E, N):
    info = plsc.get_sparse_core_info()
    NC, NS = info.num_cores, info.num_subcores
    NW = NC * NS
    n_chunks = E // CH
    steps = (n_chunks + NW - 1) // NW
    n_half = (steps + 2) // 2
    gpt = (N // 8) // NS
    rpt = gpt * 8
    rlast = N - (NS - 1) * rpt
    mesh = plsc.VectorSubcoreMesh(core_axis_name="c", subcore_axis_name="s")

    @functools.partial(
        pl.kernel,
        out_type=jax.ShapeDtypeStruct((NC, N, H), F32),
        mesh=mesh,
        scratch_types=[
            pltpu.VMEM((CH,), jnp.int32), pltpu.VMEM((CH,), jnp.int32),
            pltpu.SemaphoreType.DMA, pltpu.SemaphoreType.DMA,
            pltpu.VMEM((CH, H), F32), pltpu.VMEM((CH, H), F32),
            pltpu.SemaphoreType.DMA, pltpu.SemaphoreType.DMA,
            pltpu.VMEM_SHARED((N, H), F32),
        ],
    )
    def scatter_k(m_h, col_h, z128_h, agg_o,
                  ix0, ix1, si0, si1, mb0, mb1, sm0, sm1, agg_sh):
        IX, SI = (ix0, ix1), (si0, si1)
        MB, SM = (mb0, mb1), (sm0, sm1)
        cid = lax.axis_index("c")
        sid = lax.axis_index("s")
        wid = sid * NC + cid
        r0 = sid * rpt

        @pl.when(sid < NS - 1)
        def _():
            pltpu.sync_copy(z128_h.at[pl.ds(r0, rpt)],
                            agg_sh.at[pl.ds(r0, rpt)])

        @pl.when(sid == NS - 1)
        def _():
            pltpu.sync_copy(z128_h.at[pl.ds(r0, rlast)],
                            agg_sh.at[pl.ds(r0, rlast)])

        plsc.subcore_barrier()

        def issue(k, b):
            c = k * NW + wid

            @pl.when(c < n_chunks)
            def _():
                base = c * CH
                pltpu.async_copy(col_h.at[pl.ds(base, CH)], IX[b], SI[b])
                pltpu.async_copy(m_h.at[pl.ds(base, CH)], MB[b], SM[b])

        def drain(k, b):
            c = k * NW + wid

            @pl.when((k >= 0) & (c < n_chunks))
            def _():
                base = c * CH
                pltpu.make_async_copy(
                    col_h.at[pl.ds(base, CH)], IX[b], SI[b]).wait()
                pltpu.make_async_copy(
                    m_h.at[pl.ds(base, CH)], MB[b], SM[b]).wait()
                pltpu.sync_copy(MB[b], agg_sh.at[IX[b]], add=True)

        def pair(k2, carry):
            for b2 in range(2):
                k = 2 * k2 + b2
                issue(k, b2)
                drain(k - 1, 1 - b2)
            return carry

        lax.fori_loop(0, n_half, pair, 0)
        plsc.subcore_barrier()

        @pl.when(sid < NS - 1)
        def _():
            pltpu.sync_copy(agg_sh.at[pl.ds(r0, rpt)],
                            agg_o.at[cid, pl.ds(r0, rpt)])

        @pl.when(sid == NS - 1)
        def _():
            pltpu.sync_copy(agg_sh.at[pl.ds(r0, rlast)],
                            agg_o.at[cid, pl.ds(r0, rlast)])

    return scatter_k


def _make_tc_embed(N):
    def body(h0, We, be, W1r, W1c, b1f, h_o, pr_o, pc_o):
        h1 = _mm(h0[...], We[...]) + be[...]
        pr_o[...] = _mm(h1, W1r[...]) + b1f[...]
        pc_o[...] = _mm(h1, W1c[...])
        h_o[...] = h1

    grid = (N // BN,)
    return pl.pallas_call(
        body,
        grid=grid,
        in_specs=[
            pl.BlockSpec((BN, H), lambda i: (i, 0)),
            _full((H, H)), _full((1, H)), _full((H, H)), _full((H, H)),
            _full((1, H)),
        ],
        out_specs=[
            pl.BlockSpec((BN, H), lambda i: (i, 0)),
            pl.BlockSpec((BN, H), lambda i: (i, 0)),
            pl.BlockSpec((BN, H), lambda i: (i, 0)),
        ],
        out_shape=[
            jax.ShapeDtypeStruct((N, H), F32),
            jax.ShapeDtypeStruct((N, H), F32),
            jax.ShapeDtypeStruct((N, H), F32),
        ],
    )


def _make_tc_edge(E):
    def body(ts, rel4, wd, W2, b2, C1, c1b, C2p, m_o, cv_o):
        rel = rel4[...][:, :3]
        dist = jnp.sqrt(jnp.sum(rel * rel, axis=1, keepdims=True))
        bf = jnp.bfloat16
        t = ts[...] + dist.astype(bf).astype(F32) * wd[...].astype(bf).astype(F32)
        m = _silu(_mm(_silu(t), W2[...]) + b2[...])
        q = _silu(_mm(m, C1[...]) + c1b[...])
        cm = jnp.tanh(_mm(q, C2p[...])[:, :1])
        cv = cm * (rel / (dist + 1e-8))
        cv_o[...] = jnp.concatenate(
            [cv, jnp.ones((BE, 1), F32), jnp.zeros((BE, H - 4), F32)], axis=1)
        m_o[...] = m

    grid = (E // BE,)
    return pl.pallas_call(
        body,
        grid=grid,
        in_specs=[
            pl.BlockSpec((BE, H), lambda i: (i, 0)),
            pl.BlockSpec((BE, 4), lambda i: (i, 0)),
            _full((1, H)), _full((H, H)), _full((1, H)),
            _full((H, H)), _full((1, H)), _full((H, 8)),
        ],
        out_specs=[
            pl.BlockSpec((BE, H), lambda i: (i, 0)),
            pl.BlockSpec((BE, H), lambda i: (i, 0)),
        ],
        out_shape=[
            jax.ShapeDtypeStruct((E, H), F32),
            jax.ShapeDtypeStruct((E, H), F32),
        ],
    )


def _make_tc_node(N):
    def body(h, agg0, agg1, cu0, cu1, x4,
             A1h, A1a, ab1, A2, ab2, g, bln, W1r, W1c, b1f,
             h_o, x_o, pr_o, pc_o):
        hh = h[...]
        agg = agg0[...] + agg1[...]
        nu = _silu(_mm(hh, A1h[...]) + _mm(agg, A1a[...]) + ab1[...])
        hu = hh + _mm(nu, A2[...]) + ab2[...]
        mu = jnp.mean(hu, axis=1, keepdims=True)
        var = jnp.mean((hu - mu) ** 2, axis=1, keepdims=True)
        hn = (hu - mu) / jnp.sqrt(var + 1e-5) * g[...] + bln[...]
        c0 = cu0[...]
        c1 = cu1[...]
        cu = c0[:, :3] + c1[:, :3]
        deg = c0[:, 3:4] + c1[:, 3:4]
        xn = x4[...][:, :3] + cu / (deg + 1.0)
        pr_o[...] = _mm(hn, W1r[...]) + b1f[...]
        pc_o[...] = _mm(hn, W1c[...])
        h_o[...] = hn
        x_o[...] = jnp.concatenate([xn, jnp.zeros((BN, 1), F32)], axis=1)

    grid = (N // BN,)
    return pl.pallas_call(
        body,
        grid=grid,
        in_specs=[
            pl.BlockSpec((BN, H), lambda i: (i, 0)),
            pl.BlockSpec((BN, H), lambda i: (i, 0)),
            pl.BlockSpec((BN, H), lambda i: (i, 0)),
            pl.BlockSpec((BN, H), lambda i: (i, 0)),
            pl.BlockSpec((BN, H), lambda i: (i, 0)),
            pl.BlockSpec((BN, 4), lambda i: (i, 0)),
            _full((H, H)), _full((H, H)), _full((1, H)),
            _full((H, H)), _full((1, H)), _full((1, H)), _full((1, H)),
            _full((H, H)), _full((H, H)), _full((1, H)),
        ],
        out_specs=[
            pl.BlockSpec((BN, H), lambda i: (i, 0)),
            pl.BlockSpec((BN, 4), lambda i: (i, 0)),
            pl.BlockSpec((BN, H), lambda i: (i, 0)),
            pl.BlockSpec((BN, H), lambda i: (i, 0)),
        ],
        out_shape=[
            jax.ShapeDtypeStruct((N, H), F32),
            jax.ShapeDtypeStruct((N, 4), F32),
            jax.ShapeDtypeStruct((N, H), F32),
            jax.ShapeDtypeStruct((N, H), F32),
        ],
    )


def _make_tc_eps_edge(E):
    def body(ts, rel4, wrel4, wd, W2, b2, Ce1, ce1b, Ce2p, ce2bp, es_o):
        r4 = rel4[...]
        rel = r4[:, :3]
        dist = jnp.sqrt(jnp.sum(rel * rel, axis=1, keepdims=True))
        bf = jnp.bfloat16
        t = (ts[...] + _mm(r4, wrel4[...])
             + dist.astype(bf).astype(F32) * wd[...].astype(bf).astype(F32))
        em = _silu(_mm(_silu(t), W2[...]) + b2[...])
        q = _silu(_mm(em, Ce1[...]) + ce1b[...])
        es_o[...] = _mm(q, Ce2p[...]) + ce2bp[...]

    grid = (E // BE,)
    return pl.pallas_call(
        body,
        grid=grid,
        in_specs=[
            pl.BlockSpec((BE, H), lambda i: (i, 0)),
            pl.BlockSpec((BE, 4), lambda i: (i, 0)),
            _full((4, H)), _full((1, H)),
            _full((H, H)), _full((1, H)),
            _full((H, H)), _full((1, H)), _full((H, H)), _full((1, H)),
        ],
        out_specs=[pl.BlockSpec((BE, H), lambda i: (i, 0))],
        out_shape=[jax.ShapeDtypeStruct((E, H), F32)],
    )


def _make_tc_head(N):
    def body(h, x4, ep0, ep1, Wh, Whx4, bh1, Wh2p, bh2p, eps_o):
        q = _silu(_mm(h[...], Wh[...]) + _mm(x4[...], Whx4[...])
                  + bh1[...])
        local = _mm(q, Wh2p[...]) + bh2p[...]
        eps_o[...] = ep0[...][:, :8] + ep1[...][:, :8] + local

    grid = (N // BN,)
    return pl.pallas_call(
        body,
        grid=grid,
        in_specs=[
            pl.BlockSpec((BN, H), lambda i: (i, 0)),
            pl.BlockSpec((BN, 4), lambda i: (i, 0)),
            pl.BlockSpec((BN, H), lambda i: (i, 0)),
            pl.BlockSpec((BN, H), lambda i: (i, 0)),
            _full((H, H)), _full((4, H)), _full((1, H)),
            _full((H, 8)), _full((1, 8)),
        ],
        out_specs=[pl.BlockSpec((BN, 8), lambda i: (i, 0))],
        out_shape=[jax.ShapeDtypeStruct((N, 8), F32)],
    )


def _pad_cols(w, width):
    return jnp.concatenate(
        [w, jnp.zeros((w.shape[0], width - w.shape[1]), F32)], axis=1)


def _pad_rows(w, height):
    return jnp.concatenate(
        [w, jnp.zeros((height - w.shape[0], w.shape[1]), F32)], axis=0)


def kernel(h, x, edge_index, params):
    N = h.shape[0]
    E = edge_index.shape[1]
    row = edge_index[0]
    col = edge_index[1]
    x4 = jnp.concatenate([x, jnp.zeros((N, 1), F32)], axis=1)
    z128 = jnp.zeros((N, H), F32)

    embed = _make_tc_embed(N)
    gather = _make_sc_gather(E, N)
    relk = _make_sc_rel(E, N)
    edge = _make_tc_edge(E)
    scatter_m = _make_sc_scatter_m(E, N)
    node = _make_tc_node(N)
    eps_edge = _make_tc_eps_edge(E)
    head = _make_tc_head(N)

    lps = params["layers"]

    def msg_w1(l):
        W1 = lps[l]["message_mlp"][0]["W"]
        return W1[:H], W1[H:2 * H]

    W1e = params["eps_message_mlp"][0]["W"]

    W1r0, W1c0 = msg_w1(0)
    h1, pr, pc = embed(h,
                       params["embedding_in"]["W"],
                       _row2(params["embedding_in"]["b"]),
                       W1r0, W1c0,
                       _row2(lps[0]["message_mlp"][0]["b"]))

    xcur4 = x4
    for l in range(4):
        lp = lps[l]
        W1 = lp["message_mlp"][0]["W"]
        ts = gather(pr, pc, row, col)
        rel4 = relk(xcur4, row, col)
        m, cv8 = edge(ts, rel4,
                      W1[2 * H:2 * H + 1],
                      lp["message_mlp"][1]["W"],
                      _row2(lp["message_mlp"][1]["b"]),
                      lp["coord_mlp"][0]["W"],
                      _row2(lp["coord_mlp"][0]["b"]),
                      _pad_cols(lp["coord_mlp"][1]["W"], 8))
        aggp = scatter_m(m, col, z128)
        cup = scatter_m(cv8, col, z128)
        A1 = lp["node_update_mlp"][0]["W"]
        if l < 3:
            W1rn, W1cn = msg_w1(l + 1)
            b1n = lps[l + 1]["message_mlp"][0]["b"]
        else:
            W1rn, W1cn = W1e[:H], W1e[H:2 * H]
            b1n = params["eps_message_mlp"][0]["b"]
        h1, xcur4, pr, pc = node(h1, aggp[0], aggp[1], cup[0], cup[1], xcur4,
                                 A1[:H], A1[H:],
                                 _row2(lp["node_update_mlp"][0]["b"]),
                                 lp["node_update_mlp"][1]["W"],
                                 _row2(lp["node_update_mlp"][1]["b"]),
                                 _row2(lp["ln"]["g"]), _row2(lp["ln"]["b"]),
                                 W1rn, W1cn, _row2(b1n))

    ts = gather(pr, pc, row, col)
    rel4 = relk(xcur4, row, col)
    es8 = eps_edge(ts, rel4,
                   _pad_rows(W1e[2 * H:2 * H + 3], 4),
                   W1e[2 * H + 3:2 * H + 4],
                   params["eps_message_mlp"][1]["W"],
                   _row2(params["eps_message_mlp"][1]["b"]),
                   params["eps_coord_mlp"][0]["W"],
                   _row2(params["eps_coord_mlp"][0]["b"]),
                   _pad_cols(params["eps_coord_mlp"][1]["W"], H),
                   _row2(_pad_cols(params["eps_coord_mlp"][1]["b"]
                                   .reshape(1, 3), H)[0]))
    epp = scatter_m(es8[0], col, z128)
    Wh1 = params["eps_head"][0]["W"]
    (eps8,) = head(h1, xcur4, epp[0], epp[1],
                   Wh1[:H], _pad_rows(Wh1[H:H + 3], 4),
                   _row2(params["eps_head"][0]["b"]),
                   _pad_cols(params["eps_head"][1]["W"], 8),
                   _row2(_pad_cols(params["eps_head"][1]["b"]
                                   .reshape(1, 3), 8)[0]))

    return (h1, xcur4[:, :3], eps8[:, :3])


# speedup vs baseline: 1.0838x; 1.0838x over previous
import functools

import jax
import jax.numpy as jnp
from jax import lax
from jax.experimental import pallas as pl
from jax.experimental.pallas import tpu as pltpu
from jax.experimental.pallas import tpu_sc as plsc

H = 128
BE = 3200
BN = 1000
CH = 128
CHG = 80
RB = 2000
F32 = jnp.float32


def _mm(a, b):
    return lax.dot_general(a.astype(jnp.bfloat16), b.astype(jnp.bfloat16),
                           (((1,), (0,)), ((), ())),
                           preferred_element_type=F32,
                           precision=lax.Precision.DEFAULT)


def _silu(x):
    return 0.5 * x * (1.0 + jnp.tanh(0.5 * x))


def _full(shape):
    return pl.BlockSpec(shape, lambda i: (0,) * len(shape))


def _row2(p):
    return p.reshape(1, -1)


def _make_sc_gather(E, N):
    info = plsc.get_sparse_core_info()
    NC, NS = info.num_cores, info.num_subcores
    NW = NC * NS
    ept = E // NW
    n_chunks = ept // CHG
    mesh = plsc.VectorSubcoreMesh(core_axis_name="c", subcore_axis_name="s")

    @functools.partial(
        pl.kernel,
        out_type=jax.ShapeDtypeStruct((E, H), F32),
        mesh=mesh,
        scratch_types=[
            pltpu.VMEM((ept,), jnp.int32), pltpu.VMEM((ept,), jnp.int32),
            pltpu.VMEM((CHG, H), F32), pltpu.VMEM((CHG, H), F32),
            pltpu.VMEM((CHG, H), F32), pltpu.VMEM((CHG, H), F32),
            pltpu.SemaphoreType.DMA, pltpu.SemaphoreType.DMA,
            pltpu.SemaphoreType.DMA, pltpu.SemaphoreType.DMA,
            pltpu.SemaphoreType.DMA, pltpu.SemaphoreType.DMA,
        ],
    )
    def gather_k(tab_r, tab_c, row_h, col_h, t_o,
                 idxr, idxc, br0, bc0, br1, bc1,
                 sgr0, sgc0, sgr1, sgc1, so0, so1):
        wid = lax.axis_index("s") * NC + lax.axis_index("c")
        e0 = wid * ept
        BRb, BCb = (br0, br1), (bc0, bc1)
        SGR, SGC, SO = (sgr0, sgr1), (sgc0, sgc1), (so0, so1)
        pltpu.sync_copy(row_h.at[pl.ds(e0, ept)], idxr)
        pltpu.sync_copy(col_h.at[pl.ds(e0, ept)], idxc)

        def issue(k, b):
            @pl.when((k >= 2) & (k - 2 < n_chunks))
            def _():
                pltpu.make_async_copy(
                    BRb[b], t_o.at[pl.ds(0, CHG)], SO[b]).wait()

            @pl.when(k < n_chunks)
            def _():
                off = k * CHG
                pltpu.async_copy(
                    tab_r.at[idxr.at[pl.ds(off, CHG)]], BRb[b], SGR[b])
                pltpu.async_copy(
                    tab_c.at[idxc.at[pl.ds(off, CHG)]], BCb[b], SGC[b])

        def drain(k, b):
            @pl.when((k >= 0) & (k < n_chunks))
            def _():
                off = k * CHG
                pltpu.make_async_copy(
                    tab_r.at[idxr.at[pl.ds(off, CHG)]], BRb[b], SGR[b]).wait()
                pltpu.make_async_copy(
                    tab_c.at[idxc.at[pl.ds(off, CHG)]], BCb[b], SGC[b]).wait()

                def rbody(r, carry):
                    for cix in range(H // 16):
                        sl = pl.ds(cix * 16, 16)
                        BRb[b][r, sl] = BRb[b][r, sl] + BCb[b][r, sl]
                    return carry

                lax.fori_loop(0, CHG, rbody, 0)
                pltpu.async_copy(BRb[b], t_o.at[pl.ds(e0 + off, CHG)], SO[b])

        def pair(k2, carry):
            for b2 in range(2):
                k = 2 * k2 + b2
                issue(k, b2)
                drain(k - 1, 1 - b2)
            return carry

        lax.fori_loop(0, (n_chunks + 4) // 2, pair, 0)

    return gather_k


def _make_sc_rel(E, N):
    info = plsc.get_sparse_core_info()
    NC, NS = info.num_cores, info.num_subcores
    NW = NC * NS
    ept = E // NW
    n_sup = ept // RB
    nv = RB // 16
    mesh = plsc.VectorSubcoreMesh(core_axis_name="c", subcore_axis_name="s")

    @functools.partial(
        pl.kernel,
        out_type=jax.ShapeDtypeStruct((E, 4), F32),
        mesh=mesh,
        compiler_params=pltpu.CompilerParams(use_tc_tiling_on_sc=False,
                                             needs_layout_passes=False),
        scratch_types=[
            pltpu.VMEM((N, 4), F32),
            pltpu.VMEM((RB,), jnp.int32), pltpu.VMEM((RB,), jnp.int32),
            pltpu.VMEM((RB, 4), F32), pltpu.VMEM((RB, 4), F32),
            pltpu.SemaphoreType.DMA, pltpu.SemaphoreType.DMA,
        ],
    )
    def rel_k(x4_h, row_h, col_h, rel_o, xtab, idxr, idxc, rb0, rb1, so0, so1):
        wid = lax.axis_index("s") * NC + lax.axis_index("c")
        e0 = wid * ept
        RBb, SO = (rb0, rb1), (so0, so1)
        pltpu.sync_copy(x4_h, xtab)
        lane = lax.iota(jnp.int32, 16)

        for s in range(n_sup):
            b = s % 2
            if s >= 2:
                pltpu.make_async_copy(
                    RBb[b], rel_o.at[pl.ds(0, RB)], SO[b]).wait()
            pltpu.sync_copy(row_h.at[pl.ds(e0 + s * RB, RB)], idxr)
            pltpu.sync_copy(col_h.at[pl.ds(e0 + s * RB, RB)], idxc)

            def vbody(v, carry):
                off = v * 16
                ir = idxr[pl.ds(off, 16)]
                ic = idxc[pl.ds(off, 16)]
                lrow = v * 16 + lane
                for j in range(3):
                    js = jnp.full((16,), j, jnp.int32)
                    xr = plsc.load_gather(xtab, [ir, js])
                    xc = plsc.load_gather(xtab, [ic, js])
                    plsc.store_scatter(RBb[b], [lrow, js], xr - xc)
                plsc.store_scatter(RBb[b], [lrow, jnp.full((16,), 3, jnp.int32)],
                                   jnp.zeros((16,), F32))
                return carry

            lax.fori_loop(0, nv, vbody, 0)
            pltpu.async_copy(RBb[b], rel_o.at[pl.ds(e0 + s * RB, RB)], SO[b])

        for s in (n_sup - 2, n_sup - 1):
            pltpu.make_async_copy(
                RBb[s % 2], rel_o.at[pl.ds(0, RB)], SO[s % 2]).wait()

    return rel_k


def _make_sc_scatter_m(E, N):
    info = plsc.get_sparse_core_info()
    NC, NS = info.num_cores, info.num_subcores
    NW = NC * NS
    n_chunks = E // CH
    steps = (n_chunks + NW - 1) // NW
    n_half = (steps + 2) // 2
    gpt = (N // 8) // NS
    rpt = gpt * 8
    rlast = N - (NS - 1) * rpt
    mesh = plsc.VectorSubcoreMesh(core_axis_name="c", subcore_axis_name="s")

    @functools.partial(
        pl.kernel,
        out_type=jax.ShapeDtypeStruct((NC, N, H), F32),
        mesh=mesh,
        scratch_types=[
            pltpu.VMEM((CH,), jnp.int32), pltpu.VMEM((CH,), jnp.int32),
            pltpu.SemaphoreType.DMA, pltpu.SemaphoreType.DMA,
            pltpu.VMEM((CH, H), F32), pltpu.VMEM((CH, H), F32),
            pltpu.SemaphoreType.DMA, pltpu.SemaphoreType.DMA,
            pltpu.VMEM_SHARED((N, H), F32),
        ],
    )
    def scatter_k(m_h, col_h, z128_h, agg_o,
                  ix0, ix1, si0, si1, mb0, mb1, sm0, sm1, agg_sh):
        IX, SI = (ix0, ix1), (si0, si1)
        MB, SM = (mb0, mb1), (sm0, sm1)
        cid = lax.axis_index("c")
        sid = lax.axis_index("s")
        wid = sid * NC + cid
        r0 = sid * rpt

        @pl.when(sid < NS - 1)
        def _():
            pltpu.sync_copy(z128_h.at[pl.ds(r0, rpt)],
                            agg_sh.at[pl.ds(r0, rpt)])

        @pl.when(sid == NS - 1)
        def _():
            pltpu.sync_copy(z128_h.at[pl.ds(r0, rlast)],
                            agg_sh.at[pl.ds(r0, rlast)])

        plsc.subcore_barrier()

        def issue(k, b):
            c = k * NW + wid

            @pl.when(c < n_chunks)
            def _():
                base = c * CH
                pltpu.async_copy(col_h.at[pl.ds(base, CH)], IX[b], SI[b])
                pltpu.async_copy(m_h.at[pl.ds(base, CH)], MB[b], SM[b])

        def drain(k, b):
            c = k * NW + wid

            @pl.when((k >= 0) & (c < n_chunks))
            def _():
                base = c * CH
                pltpu.make_async_copy(
                    col_h.at[pl.ds(base, CH)], IX[b], SI[b]).wait()
                pltpu.make_async_copy(
                    m_h.at[pl.ds(base, CH)], MB[b], SM[b]).wait()
                pltpu.sync_copy(MB[b], agg_sh.at[IX[b]], add=True)

        def pair(k2, carry):
            for b2 in range(2):
                k = 2 * k2 + b2
                issue(k, b2)
                drain(k - 1, 1 - b2)
            return carry

        lax.fori_loop(0, n_half, pair, 0)
        plsc.subcore_barrier()

        @pl.when(sid < NS - 1)
        def _():
            pltpu.sync_copy(agg_sh.at[pl.ds(r0, rpt)],
                            agg_o.at[cid, pl.ds(r0, rpt)])

        @pl.when(sid == NS - 1)
        def _():
            pltpu.sync_copy(agg_sh.at[pl.ds(r0, rlast)],
                            agg_o.at[cid, pl.ds(r0, rlast)])

    return scatter_k


def _make_tc_embed(N):
    def body(h0, We, be, W1r, W1c, b1f, h_o, pr_o, pc_o):
        h1 = _mm(h0[...], We[...]) + be[...]
        pr_o[...] = _mm(h1, W1r[...]) + b1f[...]
        pc_o[...] = _mm(h1, W1c[...])
        h_o[...] = h1

    grid = (N // BN,)
    return pl.pallas_call(
        body,
        grid=grid,
        in_specs=[
            pl.BlockSpec((BN, H), lambda i: (i, 0)),
            _full((H, H)), _full((1, H)), _full((H, H)), _full((H, H)),
            _full((1, H)),
        ],
        out_specs=[
            pl.BlockSpec((BN, H), lambda i: (i, 0)),
            pl.BlockSpec((BN, H), lambda i: (i, 0)),
            pl.BlockSpec((BN, H), lambda i: (i, 0)),
        ],
        out_shape=[
            jax.ShapeDtypeStruct((N, H), F32),
            jax.ShapeDtypeStruct((N, H), F32),
            jax.ShapeDtypeStruct((N, H), F32),
        ],
    )


def _make_tc_edge(E):
    def body(ts, rel4, wd, W2, b2, C1, c1b, C2p, m_o, cv_o):
        rel = rel4[...][:, :3]
        dist = jnp.sqrt(jnp.sum(rel * rel, axis=1, keepdims=True))
        bf = jnp.bfloat16
        t = ts[...] + dist.astype(bf).astype(F32) * wd[...].astype(bf).astype(F32)
        m = _silu(_mm(_silu(t), W2[...]) + b2[...])
        q = _silu(_mm(m, C1[...]) + c1b[...])
        cm = jnp.tanh(_mm(q, C2p[...])[:, :1])
        cv = cm * (rel / (dist + 1e-8))
        cv_o[...] = jnp.concatenate(
            [cv, jnp.ones((BE, 1), F32), jnp.zeros((BE, H - 4), F32)], axis=1)
        m_o[...] = m

    grid = (E // BE,)
    return pl.pallas_call(
        body,
        grid=grid,
        in_specs=[
            pl.BlockSpec((BE, H), lambda i: (i, 0)),
            pl.BlockSpec((BE, 4), lambda i: (i, 0)),
            _full((1, H)), _full((H, H)), _full((1, H)),
            _full((H, H)), _full((1, H)), _full((H, 8)),
        ],
        out_specs=[
            pl.BlockSpec((BE, H), lambda i: (i, 0)),
            pl.BlockSpec((BE, H), lambda i: (i, 0)),
        ],
        out_shape=[
            jax.ShapeDtypeStruct((E, H), F32),
            jax.ShapeDtypeStruct((E, H), F32),
        ],
    )


def _make_tc_node(N):
    def body(h, agg0, agg1, cu0, cu1, x4,
             A1h, A1a, ab1, A2, ab2, g, bln, W1r, W1c, b1f,
             h_o, x_o, pr_o, pc_o):
        hh = h[...]
        agg = agg0[...] + agg1[...]
        nu = _silu(_mm(hh, A1h[...]) + _mm(agg, A1a[...]) + ab1[...])
        hu = hh + (_mm(nu, A2[...]) + ab2[...])
        mu = jnp.mean(hu, axis=1, keepdims=True)
        var = jnp.mean((hu - mu) ** 2, axis=1, keepdims=True)
        hn = (hu - mu) / jnp.sqrt(var + 1e-5) * g[...] + bln[...]
        c0 = cu0[...]
        c1 = cu1[...]
        cu = c0[:, :3] + c1[:, :3]
        deg = c0[:, 3:4] + c1[:, 3:4]
        xn = x4[...][:, :3] + cu / (deg + 1.0)
        pr_o[...] = _mm(hn, W1r[...]) + b1f[...]
        pc_o[...] = _mm(hn, W1c[...])
        h_o[...] = hn
        x_o[...] = jnp.concatenate([xn, jnp.zeros((BN, 1), F32)], axis=1)

    grid = (N // BN,)
    return pl.pallas_call(
        body,
        grid=grid,
        in_specs=[
            pl.BlockSpec((BN, H), lambda i: (i, 0)),
            pl.BlockSpec((BN, H), lambda i: (i, 0)),
            pl.BlockSpec((BN, H), lambda i: (i, 0)),
            pl.BlockSpec((BN, H), lambda i: (i, 0)),
            pl.BlockSpec((BN, H), lambda i: (i, 0)),
            pl.BlockSpec((BN, 4), lambda i: (i, 0)),
            _full((H, H)), _full((H, H)), _full((1, H)),
            _full((H, H)), _full((1, H)), _full((1, H)), _full((1, H)),
            _full((H, H)), _full((H, H)), _full((1, H)),
        ],
        out_specs=[
            pl.BlockSpec((BN, H), lambda i: (i, 0)),
            pl.BlockSpec((BN, 4), lambda i: (i, 0)),
            pl.BlockSpec((BN, H), lambda i: (i, 0)),
            pl.BlockSpec((BN, H), lambda i: (i, 0)),
        ],
        out_shape=[
            jax.ShapeDtypeStruct((N, H), F32),
            jax.ShapeDtypeStruct((N, 4), F32),
            jax.ShapeDtypeStruct((N, H), F32),
            jax.ShapeDtypeStruct((N, H), F32),
        ],
    )


def _make_tc_eps_edge(E):
    def body(ts, rel4, wrel4, wd, W2, b2, Ce1, ce1b, Ce2p, ce2bp, es_o):
        r4 = rel4[...]
        rel = r4[:, :3]
        dist = jnp.sqrt(jnp.sum(rel * rel, axis=1, keepdims=True))
        bf = jnp.bfloat16
        t = (ts[...] + _mm(r4, wrel4[...])
             + dist.astype(bf).astype(F32) * wd[...].astype(bf).astype(F32))
        em = _silu(_mm(_silu(t), W2[...]) + b2[...])
        q = _silu(_mm(em, Ce1[...]) + ce1b[...])
        es_o[...] = _mm(q, Ce2p[...]) + ce2bp[...]

    grid = (E // BE,)
    return pl.pallas_call(
        body,
        grid=grid,
        in_specs=[
            pl.BlockSpec((BE, H), lambda i: (i, 0)),
            pl.BlockSpec((BE, 4), lambda i: (i, 0)),
            _full((4, H)), _full((1, H)),
            _full((H, H)), _full((1, H)),
            _full((H, H)), _full((1, H)), _full((H, H)), _full((1, H)),
        ],
        out_specs=[pl.BlockSpec((BE, H), lambda i: (i, 0))],
        out_shape=[jax.ShapeDtypeStruct((E, H), F32)],
    )


def _make_tc_head(N):
    def body(h, x4, ep0, ep1, Wh, Whx4, bh1, Wh2p, bh2p, eps_o):
        q = _silu(_mm(h[...], Wh[...]) + _mm(x4[...], Whx4[...])
                  + bh1[...])
        local = _mm(q, Wh2p[...]) + bh2p[...]
        eps_o[...] = ep0[...][:, :8] + ep1[...][:, :8] + local

    grid = (N // BN,)
    return pl.pallas_call(
        body,
        grid=grid,
        in_specs=[
            pl.BlockSpec((BN, H), lambda i: (i, 0)),
            pl.BlockSpec((BN, 4), lambda i: (i, 0)),
            pl.BlockSpec((BN, H), lambda i: (i, 0)),
            pl.BlockSpec((BN, H), lambda i: (i, 0)),
            _full((H, H)), _full((4, H)), _full((1, H)),
            _full((H, 8)), _full((1, 8)),
        ],
        out_specs=[pl.BlockSpec((BN, 8), lambda i: (i, 0))],
        out_shape=[jax.ShapeDtypeStruct((N, 8), F32)],
    )


def _pad_cols(w, width):
    return jnp.concatenate(
        [w, jnp.zeros((w.shape[0], width - w.shape[1]), F32)], axis=1)


def _pad_rows(w, height):
    return jnp.concatenate(
        [w, jnp.zeros((height - w.shape[0], w.shape[1]), F32)], axis=0)


def kernel(h, x, edge_index, params):
    N = h.shape[0]
    E = edge_index.shape[1]
    row = edge_index[0]
    col = edge_index[1]
    x4 = jnp.concatenate([x, jnp.zeros((N, 1), F32)], axis=1)
    z128 = jnp.zeros((N, H), F32)

    embed = _make_tc_embed(N)
    gather = _make_sc_gather(E, N)
    relk = _make_sc_rel(E, N)
    edge = _make_tc_edge(E)
    scatter_m = _make_sc_scatter_m(E, N)
    node = _make_tc_node(N)
    eps_edge = _make_tc_eps_edge(E)
    head = _make_tc_head(N)

    lps = params["layers"]

    def msg_w1(l):
        W1 = lps[l]["message_mlp"][0]["W"]
        return W1[:H], W1[H:2 * H]

    W1e = params["eps_message_mlp"][0]["W"]

    W1r0, W1c0 = msg_w1(0)
    h1, pr, pc = embed(h,
                       params["embedding_in"]["W"],
                       _row2(params["embedding_in"]["b"]),
                       W1r0, W1c0,
                       _row2(lps[0]["message_mlp"][0]["b"]))

    xcur4 = x4
    for l in range(4):
        lp = lps[l]
        W1 = lp["message_mlp"][0]["W"]
        ts = gather(pr, pc, row, col)
        rel4 = relk(xcur4, row, col)
        m, cv8 = edge(ts, rel4,
                      W1[2 * H:2 * H + 1],
                      lp["message_mlp"][1]["W"],
                      _row2(lp["message_mlp"][1]["b"]),
                      lp["coord_mlp"][0]["W"],
                      _row2(lp["coord_mlp"][0]["b"]),
                      _pad_cols(lp["coord_mlp"][1]["W"], 8))
        aggp = scatter_m(m, col, z128)
        cup = scatter_m(cv8, col, z128)
        A1 = lp["node_update_mlp"][0]["W"]
        if l < 3:
            W1rn, W1cn = msg_w1(l + 1)
            b1n = lps[l + 1]["message_mlp"][0]["b"]
        else:
            W1rn, W1cn = W1e[:H], W1e[H:2 * H]
            b1n = params["eps_message_mlp"][0]["b"]
        h1, xcur4, pr, pc = node(h1, aggp[0], aggp[1], cup[0], cup[1], xcur4,
                                 A1[:H], A1[H:],
                                 _row2(lp["node_update_mlp"][0]["b"]),
                                 lp["node_update_mlp"][1]["W"],
                                 _row2(lp["node_update_mlp"][1]["b"]),
                                 _row2(lp["ln"]["g"]), _row2(lp["ln"]["b"]),
                                 W1rn, W1cn, _row2(b1n))

    ts = gather(pr, pc, row, col)
    rel4 = relk(xcur4, row, col)
    es8 = eps_edge(ts, rel4,
                   _pad_rows(W1e[2 * H:2 * H + 3], 4),
                   W1e[2 * H + 3:2 * H + 4],
                   params["eps_message_mlp"][1]["W"],
                   _row2(params["eps_message_mlp"][1]["b"]),
                   params["eps_coord_mlp"][0]["W"],
                   _row2(params["eps_coord_mlp"][0]["b"]),
                   _pad_cols(params["eps_coord_mlp"][1]["W"], H),
                   _row2(_pad_cols(params["eps_coord_mlp"][1]["b"]
                                   .reshape(1, 3), H)[0]))
    epp = scatter_m(es8[0], col, z128)
    Wh1 = params["eps_head"][0]["W"]
    (eps8,) = head(h1, xcur4, epp[0], epp[1],
                   Wh1[:H], _pad_rows(Wh1[H:H + 3], 4),
                   _row2(params["eps_head"][0]["b"]),
                   _pad_cols(params["eps_head"][1]["W"], 8),
                   _row2(_pad_cols(params["eps_head"][1]["b"]
                                   .reshape(1, 3), 8)[0]))

    return (h1, xcur4[:, :3], eps8[:, :3])
